# baseline (speedup 1.0000x reference)
import jax
import jax.numpy as jnp
from jax.experimental import pallas as pl
from jax.experimental.pallas import tpu as pltpu

_G = 4
_N = 4096
_K = 32
_R = 512
_NB = _N // _R
_STEPS_PER_STAGE = -(-(_K + 1) // _NB)


def _extract_range(k0, k1, vp, vn, b2p, b2n, s2pos, iota2, buf,
                   cntp_ref, cntn_ref, rowsp_ref, rowsn_ref, cextra_ref):
    for k in range(k0, k1):
        mp = jnp.max(vp, axis=(0, 1), keepdims=True)
        mn = jnp.max(vn, axis=(0, 1), keepdims=True)
        ip = jnp.min(jnp.where(vp == mp, iota2, _N), axis=(0, 1),
                     keepdims=True)
        im = jnp.min(jnp.where(vn == mn, iota2, _N), axis=(0, 1),
                     keepdims=True)
        hp = iota2 == ip
        hn = iota2 == im
        if k < _K:
            b2p = b2p + hp.astype(jnp.float32)
            b2n = b2n + hn.astype(jnp.float32)
            okp = jnp.any(jnp.logical_and(hp, s2pos))
            okn = jnp.logical_not(jnp.any(jnp.logical_and(hn, s2pos)))
            isp = ip[0, 0]
            isn = im[0, 0]
            bp = isp // _R
            bn = isn // _R
            cp = cntp_ref[buf, bp]
            rowsp_ref[buf, bp, cp] = isp
            cntp_ref[buf, bp] = cp + okp.astype(jnp.int32)
            cn = cntn_ref[buf, bn]
            rowsn_ref[buf, bn, cn] = isn
            cntn_ref[buf, bn] = cn + okn.astype(jnp.int32)
            vp = jnp.where(hp, -jnp.inf, vp)
            vn = jnp.where(hn, -jnp.inf, vn)
        else:
            cextra_ref[buf, 0] = ip[0, 0]
            cextra_ref[buf, 1] = im[0, 0]
    return vp, vn, b2p, b2n


def _mask_kernel(s_ref, t2c_ref, s2c_ref, t2n_ref, s2n_ref, o_ref,
                 cntp_ref, cntn_ref, rowsp_ref, rowsn_ref, cextra_ref,
                 basep_ref, basen_ref,
                 vp_ref, vn_ref, b2p_ref, b2n_ref):
    g = pl.program_id(0)
    nb = pl.program_id(1)
    iota = jax.lax.broadcasted_iota(jnp.int32, (1, _N), 1)
    iota2 = (jax.lax.broadcasted_iota(jnp.int32, (32, 128), 0) * 128
             + jax.lax.broadcasted_iota(jnp.int32, (32, 128), 1))
    gbuf = jax.lax.rem(g, 2)
    nbuf = jax.lax.rem(g + 1, 2)

    @pl.when(jnp.logical_and(g == 0, nb == 0))
    def _prologue():
        for b in range(_NB):
            cntp_ref[0, b] = 0
            cntn_ref[0, b] = 0
        t2 = t2c_ref[0, :, :]
        s2pos = s2c_ref[0, :, :] > 0.0
        z = jnp.zeros((32, 128), jnp.float32)
        _, _, b2p, b2n = _extract_range(
            0, _K + 1, t2, -t2, z, z, s2pos, iota2, 0,
            cntp_ref, cntn_ref, rowsp_ref, rowsn_ref, cextra_ref)
        basep_ref[0, :] = b2p.reshape(1, _N)[0, :]
        basen_ref[0, :] = b2n.reshape(1, _N)[0, :]

    @pl.when(jnp.logical_and(nb == 0, g < _G - 1))
    def _stage_init():
        for b in range(_NB):
            cntp_ref[nbuf, b] = 0
            cntn_ref[nbuf, b] = 0
        vp_ref[nbuf, :, :] = t2n_ref[0, :, :]
        vn_ref[nbuf, :, :] = -t2n_ref[0, :, :]
        b2p_ref[nbuf, :, :] = jnp.zeros((32, 128), jnp.float32)
        b2n_ref[nbuf, :, :] = jnp.zeros((32, 128), jnp.float32)

    for st in range(_NB):
        k0 = st * _STEPS_PER_STAGE
        k1 = min(_K + 1, k0 + _STEPS_PER_STAGE)
        if k0 >= k1:
            continue

        @pl.when(jnp.logical_and(nb == st, g < _G - 1))
        def _stage(k0=k0, k1=k1):
            s2pos = s2n_ref[0, :, :] > 0.0
            vp, vn, b2p, b2n = _extract_range(
                k0, k1, vp_ref[nbuf, :, :], vn_ref[nbuf, :, :],
                b2p_ref[nbuf, :, :], b2n_ref[nbuf, :, :], s2pos, iota2,
                nbuf, cntp_ref, cntn_ref, rowsp_ref, rowsn_ref, cextra_ref)
            if k1 == _K + 1:
                basep_ref[pl.ds(nbuf, 1), :] = b2p.reshape(1, _N)
                basen_ref[pl.ds(nbuf, 1), :] = b2n.reshape(1, _N)
            else:
                vp_ref[nbuf, :, :] = vp
                vn_ref[nbuf, :, :] = vn
                b2p_ref[nbuf, :, :] = b2p
                b2n_ref[nbuf, :, :] = b2n

    i0 = nb * _R
    s_blk = s_ref[0, pl.ds(i0, _R), :]
    basep = basep_ref[pl.ds(gbuf, 1), :]
    basen = basen_ref[pl.ds(gbuf, 1), :]
    o_ref[0, :, :] = jnp.where(s_blk > 0.0, basep, basen)

    extrap = (iota == cextra_ref[gbuf, 0]).astype(jnp.float32)
    extran = (iota == cextra_ref[gbuf, 1]).astype(jnp.float32)
    for cnt_ref, rows_ref, base, extra in (
            (cntp_ref, rowsp_ref, basep, extrap),
            (cntn_ref, rowsn_ref, basen, extran)):
        npatch = cnt_ref[gbuf, nb]

        def body(m, _, rows_ref=rows_ref, base=base, extra=extra):
            r = rows_ref[gbuf, nb, m]
            row = base - (iota == r).astype(jnp.float32) + extra
            o_ref[0, pl.ds(r - i0, 1), :] = row
            return 0

        jax.lax.fori_loop(0, npatch, body, 0)


def kernel(emb_s, emb_t, device):
    del device
    t2 = emb_t.reshape(_G, 32, 128)
    s2 = emb_s.reshape(_G, 32, 128)
    return pl.pallas_call(
        _mask_kernel,
        grid=(_G, _NB),
        in_specs=[
            pl.BlockSpec((1, _N, 1), lambda g, b: (g, 0, 0)),
            pl.BlockSpec((1, 32, 128), lambda g, b: (g, 0, 0)),
            pl.BlockSpec((1, 32, 128), lambda g, b: (g, 0, 0)),
            pl.BlockSpec((1, 32, 128), lambda g, b: ((g + 1) % _G, 0, 0)),
            pl.BlockSpec((1, 32, 128), lambda g, b: ((g + 1) % _G, 0, 0)),
        ],
        out_specs=pl.BlockSpec((1, _R, _N), lambda g, b: (g, b, 0)),
        out_shape=jax.ShapeDtypeStruct((_G, _N, _N), jnp.float32),
        scratch_shapes=[
            pltpu.SMEM((2, _NB), jnp.int32),
            pltpu.SMEM((2, _NB), jnp.int32),
            pltpu.SMEM((2, _NB, _K), jnp.int32),
            pltpu.SMEM((2, _NB, _K), jnp.int32),
            pltpu.SMEM((2, 2), jnp.int32),
            pltpu.VMEM((2, _N), jnp.float32),
            pltpu.VMEM((2, _N), jnp.float32),
            pltpu.VMEM((2, 32, 128), jnp.float32),
            pltpu.VMEM((2, 32, 128), jnp.float32),
            pltpu.VMEM((2, 32, 128), jnp.float32),
            pltpu.VMEM((2, 32, 128), jnp.float32),
        ],
    )(emb_s, t2, s2, t2, s2)


# speedup vs baseline: 1.0518x; 1.0518x over previous
import jax
import jax.numpy as jnp
from jax.experimental import pallas as pl
from jax.experimental.pallas import tpu as pltpu

_G = 4
_N = 4096
_K = 32
_R = 1024
_NB = _N // _R
_STEPS_PER_STAGE = -(-(_K + 1) // _NB)


def _extract_range(k0, k1, vp, vn, b2p, b2n, s2pos, iota2, buf,
                   cntp_ref, cntn_ref, rowsp_ref, rowsn_ref, cextra_ref):
    for k in range(k0, k1):
        mp = jnp.max(vp, axis=(0, 1), keepdims=True)
        mn = jnp.max(vn, axis=(0, 1), keepdims=True)
        ip = jnp.min(jnp.where(vp == mp, iota2, _N), axis=(0, 1),
                     keepdims=True)
        im = jnp.min(jnp.where(vn == mn, iota2, _N), axis=(0, 1),
                     keepdims=True)
        hp = iota2 == ip
        hn = iota2 == im
        if k < _K:
            b2p = b2p + hp.astype(jnp.float32)
            b2n = b2n + hn.astype(jnp.float32)
            okp = jnp.any(jnp.logical_and(hp, s2pos))
            okn = jnp.logical_not(jnp.any(jnp.logical_and(hn, s2pos)))
            isp = ip[0, 0]
            isn = im[0, 0]
            bp = isp // _R
            bn = isn // _R
            cp = cntp_ref[buf, bp]
            rowsp_ref[buf, bp, cp] = isp
            cntp_ref[buf, bp] = cp + okp.astype(jnp.int32)
            cn = cntn_ref[buf, bn]
            rowsn_ref[buf, bn, cn] = isn
            cntn_ref[buf, bn] = cn + okn.astype(jnp.int32)
            vp = jnp.where(hp, -jnp.inf, vp)
            vn = jnp.where(hn, -jnp.inf, vn)
        else:
            cextra_ref[buf, 0] = ip[0, 0]
            cextra_ref[buf, 1] = im[0, 0]
    return vp, vn, b2p, b2n


def _mask_kernel(s_ref, t2c_ref, s2c_ref, t2n_ref, s2n_ref, o_ref,
                 cntp_ref, cntn_ref, rowsp_ref, rowsn_ref, cextra_ref,
                 basep_ref, basen_ref,
                 vp_ref, vn_ref, b2p_ref, b2n_ref):
    g = pl.program_id(0)
    nb = pl.program_id(1)
    iota = jax.lax.broadcasted_iota(jnp.int32, (1, _N), 1)
    iota2 = (jax.lax.broadcasted_iota(jnp.int32, (32, 128), 0) * 128
             + jax.lax.broadcasted_iota(jnp.int32, (32, 128), 1))
    gbuf = jax.lax.rem(g, 2)
    nbuf = jax.lax.rem(g + 1, 2)

    @pl.when(jnp.logical_and(g == 0, nb == 0))
    def _prologue():
        for b in range(_NB):
            cntp_ref[0, b] = 0
            cntn_ref[0, b] = 0
        t2 = t2c_ref[0, :, :]
        s2pos = s2c_ref[0, :, :] > 0.0
        z = jnp.zeros((32, 128), jnp.float32)
        _, _, b2p, b2n = _extract_range(
            0, _K + 1, t2, -t2, z, z, s2pos, iota2, 0,
            cntp_ref, cntn_ref, rowsp_ref, rowsn_ref, cextra_ref)
        basep_ref[0, :] = b2p.reshape(1, _N)[0, :]
        basen_ref[0, :] = b2n.reshape(1, _N)[0, :]

    @pl.when(jnp.logical_and(nb == 0, g < _G - 1))
    def _stage_init():
        for b in range(_NB):
            cntp_ref[nbuf, b] = 0
            cntn_ref[nbuf, b] = 0
        vp_ref[nbuf, :, :] = t2n_ref[0, :, :]
        vn_ref[nbuf, :, :] = -t2n_ref[0, :, :]
        b2p_ref[nbuf, :, :] = jnp.zeros((32, 128), jnp.float32)
        b2n_ref[nbuf, :, :] = jnp.zeros((32, 128), jnp.float32)

    for st in range(_NB):
        k0 = st * _STEPS_PER_STAGE
        k1 = min(_K + 1, k0 + _STEPS_PER_STAGE)
        if k0 >= k1:
            continue

        @pl.when(jnp.logical_and(nb == st, g < _G - 1))
        def _stage(k0=k0, k1=k1):
            s2pos = s2n_ref[0, :, :] > 0.0
            vp, vn, b2p, b2n = _extract_range(
                k0, k1, vp_ref[nbuf, :, :], vn_ref[nbuf, :, :],
                b2p_ref[nbuf, :, :], b2n_ref[nbuf, :, :], s2pos, iota2,
                nbuf, cntp_ref, cntn_ref, rowsp_ref, rowsn_ref, cextra_ref)
            if k1 == _K + 1:
                basep_ref[pl.ds(nbuf, 1), :] = b2p.reshape(1, _N)
                basen_ref[pl.ds(nbuf, 1), :] = b2n.reshape(1, _N)
            else:
                vp_ref[nbuf, :, :] = vp
                vn_ref[nbuf, :, :] = vn
                b2p_ref[nbuf, :, :] = b2p
                b2n_ref[nbuf, :, :] = b2n

    i0 = nb * _R
    s_blk = s_ref[0, pl.ds(i0, _R), :]
    basep = basep_ref[pl.ds(gbuf, 1), :]
    basen = basen_ref[pl.ds(gbuf, 1), :]
    o_ref[0, :, :] = jnp.where(s_blk > 0.0, basep, basen)

    extrap = (iota == cextra_ref[gbuf, 0]).astype(jnp.float32)
    extran = (iota == cextra_ref[gbuf, 1]).astype(jnp.float32)
    for cnt_ref, rows_ref, base, extra in (
            (cntp_ref, rowsp_ref, basep, extrap),
            (cntn_ref, rowsn_ref, basen, extran)):
        npatch = cnt_ref[gbuf, nb]

        def body(m, _, rows_ref=rows_ref, base=base, extra=extra):
            r = rows_ref[gbuf, nb, m]
            row = base - (iota == r).astype(jnp.float32) + extra
            o_ref[0, pl.ds(r - i0, 1), :] = row
            return 0

        jax.lax.fori_loop(0, npatch, body, 0)


def kernel(emb_s, emb_t, device):
    del device
    t2 = emb_t.reshape(_G, 32, 128)
    s2 = emb_s.reshape(_G, 32, 128)
    return pl.pallas_call(
        _mask_kernel,
        grid=(_G, _NB),
        in_specs=[
            pl.BlockSpec((1, _N, 1), lambda g, b: (g, 0, 0)),
            pl.BlockSpec((1, 32, 128), lambda g, b: (g, 0, 0)),
            pl.BlockSpec((1, 32, 128), lambda g, b: (g, 0, 0)),
            pl.BlockSpec((1, 32, 128), lambda g, b: ((g + 1) % _G, 0, 0)),
            pl.BlockSpec((1, 32, 128), lambda g, b: ((g + 1) % _G, 0, 0)),
        ],
        out_specs=pl.BlockSpec((1, _R, _N), lambda g, b: (g, b, 0)),
        out_shape=jax.ShapeDtypeStruct((_G, _N, _N), jnp.float32),
        scratch_shapes=[
            pltpu.SMEM((2, _NB), jnp.int32),
            pltpu.SMEM((2, _NB), jnp.int32),
            pltpu.SMEM((2, _NB, _K), jnp.int32),
            pltpu.SMEM((2, _NB, _K), jnp.int32),
            pltpu.SMEM((2, 2), jnp.int32),
            pltpu.VMEM((2, _N), jnp.float32),
            pltpu.VMEM((2, _N), jnp.float32),
            pltpu.VMEM((2, 32, 128), jnp.float32),
            pltpu.VMEM((2, 32, 128), jnp.float32),
            pltpu.VMEM((2, 32, 128), jnp.float32),
            pltpu.VMEM((2, 32, 128), jnp.float32),
        ],
    )(emb_s, t2, s2, t2, s2)
